# baseline (speedup 1.0000x reference)
import functools

import jax
import jax.numpy as jnp
from jax import lax
from jax.experimental import pallas as pl
from jax.experimental.pallas import tpu as pltpu
from jax.experimental.pallas import tpu_sc as plsc

U = 25000
HALF = 25088
ROWS_PER_TEC = HALF // 16
NPAD = 2 * HALF
GAP = HALF - U
DIM = 64
E = 800000
SUB = 128
SB = 14
NSB = 28
EROWS_TEC = SB * NSB
EPAD = EROWS_TEC * SUB * 16
EROWS = EPAD // SUB
CCAP = SB * SUB + 16
DUMMY = U + 8
NWORK = 32

_MESH = dict(mesh=plsc.VectorSubcoreMesh(core_axis_name="c",
                                         subcore_axis_name="s"))


@functools.partial(
    pl.kernel,
    out_type=(jax.ShapeDtypeStruct((NWORK, NSB, CCAP), jnp.int32),
              jax.ShapeDtypeStruct((NWORK, NSB, CCAP), jnp.int32),
              jax.ShapeDtypeStruct((NWORK, NSB, CCAP), jnp.float32),
              jax.ShapeDtypeStruct((NWORK, NSB, 16), jnp.int32)),
    compiler_params=pltpu.CompilerParams(use_tc_tiling_on_sc=False,
                                         needs_layout_passes=False),
    scratch_types=[
        pltpu.VMEM((SB, SUB), jnp.int32),
        pltpu.VMEM((SB, SUB), jnp.int32),
        pltpu.VMEM((SB, SUB), jnp.float32),
        pltpu.VMEM((CCAP,), jnp.int32),
        pltpu.VMEM((CCAP,), jnp.int32),
        pltpu.VMEM((CCAP,), jnp.float32),
        pltpu.VMEM((NSB, 16), jnp.int32),
    ],
    **_MESH,
)
def _compact_edges(src, dst, w, osrc, odst, ow, ocnt,
                   esrc, edst, ew, csrc, cdst, cw, cnts):
    c = lax.axis_index("c")
    s = lax.axis_index("s")
    lo = c * U
    cgs = c * 16 + s
    row0 = s * EROWS_TEC

    zero16f = jnp.zeros((16,), jnp.float32)
    zero16i = jnp.zeros((16,), jnp.int32)
    dummy16 = jnp.full((16,), DUMMY, jnp.int32)
    lanes = lax.iota(jnp.int32, 16)

    def _superblock(sb, carry):
        base = row0 + sb * SB
        pltpu.sync_copy(src.at[pl.ds(base, SB)], esrc)
        pltpu.sync_copy(dst.at[pl.ds(base, SB)], edst)
        pltpu.sync_copy(w.at[pl.ds(base, SB)], ew)

        def _grp(kk, off):
            for g in range(8):
                sl = pl.ds(g * 16, 16)
                sv = esrc[kk, sl]
                sadj = jnp.where(sv >= U, sv + GAP, sv)
                dv = edst[kk, sl] - lo
                inr = (dv >= 0) & (dv < U)
                wv16 = ew[kk, sl]
                keys = jnp.where(inr, lanes, lanes + 16)
                _, sadj_c = plsc.sort_key_val(keys, sadj)
                _, dv_c = plsc.sort_key_val(keys, dv)
                _, w_c = plsc.sort_key_val(keys, wv16)
                csrc[pl.ds(off, 16)] = sadj_c
                cdst[pl.ds(off, 16)] = dv_c
                cw[pl.ds(off, 16)] = w_c
                cnt = plsc.all_reduce_population_count(inr)
                off = off + (cnt[0] if cnt.ndim else cnt)
            return off

        off = lax.fori_loop(0, SB, _grp, jnp.int32(0))

        for g in range(8):
            offg = jnp.minimum(off + g * 16, CCAP - 16)
            csrc[pl.ds(offg, 16)] = zero16i
            cdst[pl.ds(offg, 16)] = dummy16
            cw[pl.ds(offg, 16)] = zero16f
        nsub = lax.div(off + (SUB - 1), SUB)
        cnts[sb, pl.ds(0, 16)] = jnp.broadcast_to(nsub, (16,))

        pltpu.sync_copy(csrc, osrc.at[cgs, sb])
        pltpu.sync_copy(cdst, odst.at[cgs, sb])
        pltpu.sync_copy(cw, ow.at[cgs, sb])
        return carry

    lax.fori_loop(0, NSB, _superblock, 0)
    pltpu.sync_copy(cnts, ocnt.at[cgs])


@functools.partial(
    pl.kernel,
    out_type=jax.ShapeDtypeStruct((NPAD, DIM), jnp.float32),
    compiler_params=pltpu.CompilerParams(use_tc_tiling_on_sc=False),
    scratch_types=[
        pltpu.VMEM((CCAP,), jnp.int32),
        pltpu.VMEM((CCAP,), jnp.int32),
        pltpu.VMEM((CCAP,), jnp.float32),
        pltpu.VMEM((16,), jnp.int32),
        pltpu.VMEM((2, SUB), jnp.int32),
        pltpu.VMEM((2, SUB), jnp.int32),
        pltpu.VMEM((2, SUB, DIM), jnp.float32),
        pltpu.VMEM_SHARED((HALF, DIM), jnp.float32),
        pltpu.SemaphoreType.DMA,
        pltpu.SemaphoreType.DMA,
    ],
    **_MESH,
)
def _propagate(table, csrc, cdst, cw, cnt, out,
               gsrc, gdst, gw, cntv, srcadj, dstloc, rowsv, acc, gsem, ssem):
    c = lax.axis_index("c")
    s = lax.axis_index("s")
    cgs = c * 16 + s

    zero16f = jnp.zeros((16,), jnp.float32)

    def _zero_rowsv(r, carry):
        for b in range(4):
            rowsv[0, r, pl.ds(b * 16, 16)] = zero16f
        return carry

    lax.fori_loop(0, SUB, _zero_rowsv, 0)

    abase = s * ROWS_PER_TEC
    for k in range(12):
        pltpu.sync_copy(rowsv.at[0], acc.at[pl.ds(abase + k * SUB, SUB)])
    pltpu.sync_copy(rowsv.at[0].at[pl.ds(0, 32)],
                    acc.at[pl.ds(abase + 12 * SUB, 32)])
    plsc.subcore_barrier()

    def _prep(k, p):
        for g in range(8):
            sl = pl.ds(g * 16, 16)
            srcadj[p, sl] = gsrc[pl.ds(k * SUB + g * 16, 16)]
            dstloc[p, sl] = gdst[pl.ds(k * SUB + g * 16, 16)]

    def _fire_gather(p):
        pltpu.async_copy(table.at[srcadj.at[p]], rowsv.at[p], gsem)

    def _wait_gather(p):
        pltpu.make_async_copy(table.at[srcadj.at[p]], rowsv.at[p],
                              gsem).wait()

    def _fire_scatter(p):
        pltpu.async_copy(rowsv.at[p], acc.at[dstloc.at[p]], ssem, add=True)

    def _wait_scatter(p):
        pltpu.make_async_copy(rowsv.at[p], acc.at[dstloc.at[p]],
                              ssem).wait()

    def _scale(k, p):
        def body(g, carry):
            wvec = gw[pl.ds(k * SUB + g * 16, 16)]
            for e in range(16):
                ws = jnp.broadcast_to(wvec[e], (16,))
                r = g * 16 + e
                for b in range(4):
                    rowsv[p, r, pl.ds(b * 16, 16)] = (
                        rowsv[p, r, pl.ds(b * 16, 16)] * ws)
            return carry
        lax.fori_loop(0, 8, body, 0)

    def _superblock(sb, carry):
        pltpu.sync_copy(cnt.at[cgs, sb], cntv)
        nsub = cntv[pl.ds(0, 16)][0]
        pltpu.sync_copy(csrc.at[cgs, sb], gsrc)
        pltpu.sync_copy(cdst.at[cgs, sb], gdst)
        pltpu.sync_copy(cw.at[cgs, sb], gw)

        @pl.when(nsub > 0)
        def _():
            _prep(0, 0)
            _fire_gather(0)

            def _step(k, carry2):
                p = lax.rem(k, 2)
                pn = lax.rem(k + 1, 2)

                @pl.when(k + 1 < nsub)
                def _():
                    @pl.when(k >= 1)
                    def _():
                        _wait_scatter(pn)

                    _prep(k + 1, pn)
                    _fire_gather(pn)

                _wait_gather(p)
                _scale(k, p)
                _fire_scatter(p)
                return carry2

            lax.fori_loop(0, nsub, _step, 0)

            @pl.when(nsub >= 2)
            def _():
                _wait_scatter(lax.rem(nsub - 2, 2))

            _wait_scatter(lax.rem(nsub - 1, 2))

        return carry

    lax.fori_loop(0, NSB, _superblock, 0)
    plsc.subcore_barrier()

    obase = c * HALF + abase
    for k in range(12):
        pltpu.sync_copy(acc.at[pl.ds(abase + k * SUB, SUB)],
                        out.at[pl.ds(obase + k * SUB, SUB)])
    pltpu.sync_copy(acc.at[pl.ds(abase + 12 * SUB, 32)],
                    out.at[pl.ds(obase + 12 * SUB, 32)])


def kernel(user_emb, item_emb, edge_index, edge_weight):
    src = edge_index[0].astype(jnp.int32)
    dst = edge_index[1].astype(jnp.int32)
    w = edge_weight.astype(jnp.float32)
    pad = EPAD - E
    src = jnp.concatenate([src, jnp.zeros((pad,), jnp.int32)]).reshape(EROWS, SUB)
    dst = jnp.concatenate([dst, jnp.zeros((pad,), jnp.int32)]).reshape(EROWS, SUB)
    w = jnp.concatenate([w, jnp.zeros((pad,), jnp.float32)]).reshape(EROWS, SUB)
    csrc, cdst, cw, cnt = _compact_edges(src, dst, w)
    gap = jnp.zeros((GAP, DIM), jnp.float32)
    e0 = jnp.concatenate([user_emb, gap, item_emb, gap], axis=0)
    e1 = _propagate(e0, csrc, cdst, cw, cnt)
    e2 = _propagate(e1, csrc, cdst, cw, cnt)
    light = (e0 + e1 + e2) * (1.0 / 3.0)
    return light[:U], light[HALF:HALF + U]


# speedup vs baseline: 1.8783x; 1.8783x over previous
import functools

import jax
import jax.numpy as jnp
from jax import lax
from jax.experimental import pallas as pl
from jax.experimental.pallas import tpu as pltpu
from jax.experimental.pallas import tpu_sc as plsc

U = 25000
HALF = 25088
ROWS_PER_TEC = HALF // 16
NPAD = 2 * HALF
GAP = HALF - U
DIM = 64
E = 800000
SUB = 128
SB = 14
SBS_PER_TEC = 28
SUBS_PER_TEC = SB * SBS_PER_TEC
EPAD = SUBS_PER_TEC * SUB * 16
EROWS = EPAD // SUB
DUMMY = U + 8


@functools.partial(
    pl.kernel,
    out_type=jax.ShapeDtypeStruct((NPAD, DIM), jnp.float32),
    mesh=plsc.VectorSubcoreMesh(core_axis_name="c", subcore_axis_name="s"),
    compiler_params=pltpu.CompilerParams(use_tc_tiling_on_sc=False),
    scratch_types=[
        pltpu.VMEM((SB, SUB), jnp.int32),
        pltpu.VMEM((SB, SUB), jnp.int32),
        pltpu.VMEM((SB, SUB), jnp.float32),
        pltpu.VMEM((2, SUB), jnp.int32),
        pltpu.VMEM((2, SUB), jnp.int32),
        pltpu.VMEM((2, SUB), jnp.float32),
        pltpu.VMEM((2, SUB, DIM), jnp.float32),
        pltpu.VMEM_SHARED((HALF, DIM), jnp.float32),
        pltpu.SemaphoreType.DMA,
        pltpu.SemaphoreType.DMA,
    ],
)
def _propagate(table, src, dst, w, out, esrc, edst, ew, srcadj, dstloc,
               wring, rowsv, acc, gsem, ssem):
    c = lax.axis_index("c")
    s = lax.axis_index("s")
    lo = c * U

    zero16 = jnp.zeros((16,), jnp.float32)

    def _zero_rowsv(r, carry):
        for b in range(4):
            rowsv[0, r, pl.ds(b * 16, 16)] = zero16
        return carry

    lax.fori_loop(0, SUB, _zero_rowsv, 0)

    abase = s * ROWS_PER_TEC
    for k in range(12):
        pltpu.sync_copy(rowsv.at[0], acc.at[pl.ds(abase + k * SUB, SUB)])
    pltpu.sync_copy(rowsv.at[0].at[pl.ds(0, 32)],
                    acc.at[pl.ds(abase + 12 * SUB, 32)])
    plsc.subcore_barrier()

    row0 = s * SUBS_PER_TEC

    def _load_sb(sb):
        base = row0 + sb * SB
        pltpu.sync_copy(src.at[pl.ds(base, SB)], esrc)
        pltpu.sync_copy(dst.at[pl.ds(base, SB)], edst)
        pltpu.sync_copy(w.at[pl.ds(base, SB)], ew)

    def _prep(n):
        kk = lax.rem(n, SB)
        p = lax.rem(n, 2)
        for g in range(8):
            sl = pl.ds(g * 16, 16)
            sv = esrc[kk, sl]
            srcadj[p, sl] = jnp.where(sv >= U, sv + GAP, sv)
            dv = edst[kk, sl] - lo
            inr = (dv >= 0) & (dv < U)
            dstloc[p, sl] = jnp.where(inr, dv, DUMMY)
            wring[p, sl] = ew[kk, sl]

    def _fire_gather(p):
        pltpu.async_copy(table.at[srcadj.at[p]], rowsv.at[p], gsem)

    def _wait_gather(p):
        pltpu.make_async_copy(table.at[srcadj.at[p]], rowsv.at[p],
                              gsem).wait()

    def _fire_scatter(p):
        pltpu.async_copy(rowsv.at[p], acc.at[dstloc.at[p]], ssem, add=True)

    def _wait_scatter(p):
        pltpu.make_async_copy(rowsv.at[p], acc.at[dstloc.at[p]],
                              ssem).wait()

    def _scale(p):
        def body(g, carry):
            wvec = wring[p, pl.ds(g * 16, 16)]
            for e in range(16):
                ws = jnp.broadcast_to(wvec[e], (16,))
                r = g * 16 + e
                for b in range(4):
                    rowsv[p, r, pl.ds(b * 16, 16)] = (
                        rowsv[p, r, pl.ds(b * 16, 16)] * ws)
            return carry
        lax.fori_loop(0, 8, body, 0)

    _load_sb(0)
    _prep(0)
    _fire_gather(0)

    def _step(i, carry):
        nxt = i + 1
        p = lax.rem(i, 2)
        pn = lax.rem(nxt, 2)

        @pl.when(nxt < SUBS_PER_TEC)
        def _():
            @pl.when(i >= 1)
            def _():
                _wait_scatter(pn)

            @pl.when(lax.rem(nxt, SB) == 0)
            def _():
                _load_sb(lax.div(nxt, SB))

            _prep(nxt)
            _fire_gather(pn)

        _wait_gather(p)
        _scale(p)
        _fire_scatter(p)
        return carry

    lax.fori_loop(0, SUBS_PER_TEC, _step, 0)
    _wait_scatter(0)
    _wait_scatter(1)
    plsc.subcore_barrier()

    obase = c * HALF + abase
    for k in range(12):
        pltpu.sync_copy(acc.at[pl.ds(abase + k * SUB, SUB)],
                        out.at[pl.ds(obase + k * SUB, SUB)])
    pltpu.sync_copy(acc.at[pl.ds(abase + 12 * SUB, 32)],
                    out.at[pl.ds(obase + 12 * SUB, 32)])


def kernel(user_emb, item_emb, edge_index, edge_weight):
    src = edge_index[0].astype(jnp.int32)
    dst = edge_index[1].astype(jnp.int32)
    w = edge_weight.astype(jnp.float32)
    pad = EPAD - E
    src = jnp.concatenate([src, jnp.zeros((pad,), jnp.int32)]).reshape(EROWS, SUB)
    dst = jnp.concatenate([dst, jnp.zeros((pad,), jnp.int32)]).reshape(EROWS, SUB)
    w = jnp.concatenate([w, jnp.zeros((pad,), jnp.float32)]).reshape(EROWS, SUB)
    gap = jnp.zeros((GAP, DIM), jnp.float32)
    e0 = jnp.concatenate([user_emb, gap, item_emb, gap], axis=0)
    e1 = _propagate(e0, src, dst, w)
    e2 = _propagate(e1, src, dst, w)
    light = (e0 + e1 + e2) * (1.0 / 3.0)
    return light[:U], light[HALF:HALF + U]
